# Initial kernel scaffold; baseline (speedup 1.0000x reference)
import jax
import jax.numpy as jnp
from jax.experimental import pallas as pl

K = 8
W_CONST = 0.01
EPS = 1e-8
N_V = 16384
BN = 128


def _body(xa_ref, x_ref, vt_ref, R_ref, o_ref):
    d2 = jnp.dot(xa_ref[...], vt_ref[...], preferred_element_type=jnp.float32)

    m_prev = jnp.full((BN, 1), -jnp.inf, dtype=jnp.float32)
    m1 = None
    for k in range(K):
        cur = jnp.where(d2 > m_prev, d2, jnp.inf)
        m_prev = jnp.min(cur, axis=1, keepdims=True)
        if k == 0:
            m1 = m_prev
    t8 = m_prev

    inv = 1.0 / jnp.maximum(d2, EPS)
    w = jnp.where(d2 <= t8, inv, 0.0)
    onehot = jnp.where(d2 <= m1, 1.0, 0.0)

    P1 = jnp.dot(w, R_ref[...], preferred_element_type=jnp.float32)
    P2 = jnp.dot(onehot, R_ref[...], preferred_element_type=jnp.float32)

    term_knn = P1[:, 0:3]
    Wk = P1[:, 3:4]
    cnt = P2[:, 3:4]
    v1 = P2[:, 4:7] / cnt

    x = x_ref[...]
    dv = x - v1
    d2v1 = jnp.maximum(jnp.sum(dv * dv, axis=1, keepdims=True), EPS)
    term_dir = dv / (W_CONST * d2v1)
    W = Wk + 1.0 / W_CONST
    n_tilde = (term_knn + term_dir) / W
    nrm = jnp.sqrt(jnp.sum(n_tilde * n_tilde, axis=1, keepdims=True))
    nc = n_tilde / (nrm + 1e-8)
    s = jnp.sum(dv * nc, axis=1, keepdims=True)
    xc = x - s * nc
    o_ref[...] = jnp.concatenate([xc, s, nc], axis=1)


def kernel(x, vertices, vertex_normals):
    x = x.astype(jnp.float32)
    vertices = vertices.astype(jnp.float32)
    vertex_normals = vertex_normals.astype(jnp.float32)

    n = x.shape[0]
    v = vertices.shape[0]

    x2 = jnp.sum(x * x, axis=1, keepdims=True)
    ones_n = jnp.ones((n, 1), jnp.float32)
    xa = jnp.concatenate([-2.0 * x, ones_n, x2, jnp.zeros((n, 3), jnp.float32)], axis=1)
    v2 = jnp.sum(vertices * vertices, axis=1)
    vt = jnp.concatenate(
        [vertices.T, v2[None, :], jnp.ones((1, v), jnp.float32), jnp.zeros((3, v), jnp.float32)],
        axis=0,
    )
    R = jnp.concatenate(
        [vertex_normals, jnp.ones((v, 1), jnp.float32), vertices, jnp.zeros((v, 121), jnp.float32)],
        axis=1,
    )

    grid = (n // BN,)
    out = pl.pallas_call(
        _body,
        grid=grid,
        in_specs=[
            pl.BlockSpec((BN, 8), lambda i: (i, 0)),
            pl.BlockSpec((BN, 3), lambda i: (i, 0)),
            pl.BlockSpec((8, N_V), lambda i: (0, 0)),
            pl.BlockSpec((N_V, 128), lambda i: (0, 0)),
        ],
        out_specs=pl.BlockSpec((BN, 7), lambda i: (i, 0)),
        out_shape=jax.ShapeDtypeStruct((n, 7), jnp.float32),
    )(xa, x, vt, R)
    return out


# speedup vs baseline: 4.2624x; 4.2624x over previous
import jax
import jax.numpy as jnp
from jax.experimental import pallas as pl

K = 8
W_CONST = 0.01
EPS = 1e-8
N_V = 16384
BN = 128


def _body(xa_ref, x_ref, vt_ref, R_ref, o_ref):
    xa = xa_ref[...]
    vt = vt_ref[...]
    m = jnp.dot(xa, vt, preferred_element_type=jnp.float32)
    d2 = (xa[:, 3:4] + vt[4:5, :]) - 2.0 * m

    m_prev = jnp.full((BN, 1), -jnp.inf, dtype=jnp.float32)
    m1 = None
    for k in range(K):
        cur = jnp.where(d2 > m_prev, d2, jnp.inf)
        m_prev = jnp.min(cur, axis=1, keepdims=True)
        if k == 0:
            m1 = m_prev
    t8 = m_prev

    inv = 1.0 / jnp.maximum(d2, EPS)
    w = jnp.where(d2 <= t8, inv, 0.0)
    onehot = jnp.where(d2 <= m1, 1.0, 0.0)

    P1 = jnp.dot(w, R_ref[...], preferred_element_type=jnp.float32,
                 precision=jax.lax.Precision.HIGHEST)
    P2 = jnp.dot(onehot, R_ref[...], preferred_element_type=jnp.float32,
                 precision=jax.lax.Precision.HIGHEST)

    term_knn = P1[:, 0:3]
    Wk = P1[:, 3:4]
    cnt = P2[:, 3:4]
    v1 = P2[:, 4:7] / cnt

    x = x_ref[...]
    dv = x - v1
    d2v1 = jnp.maximum(jnp.sum(dv * dv, axis=1, keepdims=True), EPS)
    term_dir = dv / (W_CONST * d2v1)
    W = Wk + 1.0 / W_CONST
    n_tilde = (term_knn + term_dir) / W
    nrm = jnp.sqrt(jnp.sum(n_tilde * n_tilde, axis=1, keepdims=True))
    nc = n_tilde / (nrm + 1e-8)
    s = jnp.sum(dv * nc, axis=1, keepdims=True)
    xc = x - s * nc
    o_ref[...] = jnp.concatenate([xc, s, nc], axis=1)


def kernel(x, vertices, vertex_normals):
    x = x.astype(jnp.float32)
    vertices = vertices.astype(jnp.float32)
    vertex_normals = vertex_normals.astype(jnp.float32)

    n = x.shape[0]
    v = vertices.shape[0]

    x2 = jnp.sum(x * x, axis=1, keepdims=True)
    xa = jnp.concatenate([x, x2, jnp.zeros((n, 4), jnp.float32)], axis=1)
    v2 = jnp.sum(vertices * vertices, axis=1)
    vt = jnp.concatenate(
        [vertices.T, jnp.zeros((1, v), jnp.float32), v2[None, :], jnp.zeros((3, v), jnp.float32)],
        axis=0,
    )
    R = jnp.concatenate(
        [vertex_normals, jnp.ones((v, 1), jnp.float32), vertices, jnp.zeros((v, 121), jnp.float32)],
        axis=1,
    )

    grid = (n // BN,)
    out = pl.pallas_call(
        _body,
        grid=grid,
        in_specs=[
            pl.BlockSpec((BN, 8), lambda i: (i, 0)),
            pl.BlockSpec((BN, 3), lambda i: (i, 0)),
            pl.BlockSpec((8, N_V), lambda i: (0, 0)),
            pl.BlockSpec((N_V, 128), lambda i: (0, 0)),
        ],
        out_specs=pl.BlockSpec((BN, 7), lambda i: (i, 0)),
        out_shape=jax.ShapeDtypeStruct((n, 7), jnp.float32),
    )(xa, x, vt, R)
    return out


# speedup vs baseline: 4.3743x; 1.0262x over previous
import jax
import jax.numpy as jnp
from jax.experimental import pallas as pl

K = 8
W_CONST = 0.01
EPS = 1e-8
N_V = 16384
BN = 128


def _body(xa_ref, x_ref, vt_ref, R_ref, o_ref):
    xa = xa_ref[...]
    vt = vt_ref[...]
    m = jnp.dot(xa, vt, preferred_element_type=jnp.float32)
    d2 = (xa[:, 3:4] + vt[4:5, :]) - 2.0 * m

    m1 = jnp.min(d2, axis=1, keepdims=True)
    m_prev = m1
    for k in range(K - 1):
        cur = jnp.where(d2 > m_prev, d2, jnp.inf)
        m_prev = jnp.min(cur, axis=1, keepdims=True)
    t8 = m_prev

    inv = 1.0 / jnp.maximum(d2, EPS)
    w = jnp.where(d2 <= t8, inv, 0.0)
    onehot = jnp.where(d2 <= m1, 1.0, 0.0)

    P1 = jnp.dot(w, R_ref[...], preferred_element_type=jnp.float32,
                 precision=jax.lax.Precision.HIGHEST)
    P2 = jnp.dot(onehot, R_ref[...], preferred_element_type=jnp.float32,
                 precision=jax.lax.Precision.HIGHEST)

    term_knn = P1[:, 0:3]
    Wk = P1[:, 3:4]
    cnt = P2[:, 3:4]
    v1 = P2[:, 4:7] / cnt

    x = x_ref[...]
    dv = x - v1
    d2v1 = jnp.maximum(jnp.sum(dv * dv, axis=1, keepdims=True), EPS)
    term_dir = dv / (W_CONST * d2v1)
    W = Wk + 1.0 / W_CONST
    n_tilde = (term_knn + term_dir) / W
    nrm = jnp.sqrt(jnp.sum(n_tilde * n_tilde, axis=1, keepdims=True))
    nc = n_tilde / (nrm + 1e-8)
    s = jnp.sum(dv * nc, axis=1, keepdims=True)
    xc = x - s * nc
    o_ref[...] = jnp.concatenate([xc, s, nc], axis=1)


def kernel(x, vertices, vertex_normals):
    x = x.astype(jnp.float32)
    vertices = vertices.astype(jnp.float32)
    vertex_normals = vertex_normals.astype(jnp.float32)

    n = x.shape[0]
    v = vertices.shape[0]

    x2 = jnp.sum(x * x, axis=1, keepdims=True)
    xa = jnp.concatenate([x, x2, jnp.zeros((n, 4), jnp.float32)], axis=1)
    v2 = jnp.sum(vertices * vertices, axis=1)
    vt = jnp.concatenate(
        [vertices.T, jnp.zeros((1, v), jnp.float32), v2[None, :], jnp.zeros((3, v), jnp.float32)],
        axis=0,
    )
    R = jnp.concatenate(
        [vertex_normals, jnp.ones((v, 1), jnp.float32), vertices, jnp.zeros((v, 121), jnp.float32)],
        axis=1,
    )

    grid = (n // BN,)
    out = pl.pallas_call(
        _body,
        grid=grid,
        in_specs=[
            pl.BlockSpec((BN, 8), lambda i: (i, 0)),
            pl.BlockSpec((BN, 3), lambda i: (i, 0)),
            pl.BlockSpec((8, N_V), lambda i: (0, 0)),
            pl.BlockSpec((N_V, 128), lambda i: (0, 0)),
        ],
        out_specs=pl.BlockSpec((BN, 7), lambda i: (i, 0)),
        out_shape=jax.ShapeDtypeStruct((n, 7), jnp.float32),
    )(xa, x, vt, R)
    return out


# speedup vs baseline: 5.3585x; 1.2250x over previous
import jax
import jax.numpy as jnp
from jax.experimental import pallas as pl

K = 8
W_CONST = 0.01
EPS = 1e-8
N_V = 16384
BN = 128


def _body(xa_ref, x_ref, vt_ref, R_ref, o_ref):
    xa = xa_ref[...]
    vt = vt_ref[...]
    m = jnp.dot(xa, vt, preferred_element_type=jnp.float32)
    e = vt[4:5, :] - 2.0 * m

    h = N_V // 2
    m0 = jnp.minimum(e[:, :h], e[:, h:])
    s0 = jnp.maximum(e[:, :h], e[:, h:])
    q = h // 2
    m1_ = jnp.minimum(m0[:, :q], m0[:, q:])
    s1 = jnp.minimum(jnp.maximum(m0[:, :q], m0[:, q:]),
                     jnp.minimum(s0[:, :q], s0[:, q:]))
    r = q // 2
    m2 = jnp.minimum(m1_[:, :r], m1_[:, r:])
    s2 = jnp.minimum(jnp.maximum(m1_[:, :r], m1_[:, r:]),
                     jnp.minimum(s1[:, :r], s1[:, r:]))
    cand = jnp.concatenate([m2, s2], axis=1)

    mn = jnp.min(cand, axis=1, keepdims=True)
    m_prev = mn
    for k in range(K - 1):
        cur = jnp.where(cand > m_prev, cand, jnp.inf)
        m_prev = jnp.min(cur, axis=1, keepdims=True)
    t8 = m_prev

    d2 = e + xa[:, 3:4]
    inv = 1.0 / jnp.maximum(d2, EPS)
    w = jnp.where(e <= t8, inv, 0.0)
    onehot = jnp.where(e <= mn, 1.0, 0.0)

    P1 = jnp.dot(w, R_ref[...], preferred_element_type=jnp.float32,
                 precision=jax.lax.Precision.HIGHEST)
    P2 = jnp.dot(onehot, R_ref[...], preferred_element_type=jnp.float32,
                 precision=jax.lax.Precision.HIGHEST)

    term_knn = P1[:, 0:3]
    Wk = P1[:, 3:4]
    cnt = P2[:, 3:4]
    v1 = P2[:, 4:7] / cnt

    x = x_ref[...]
    dv = x - v1
    d2v1 = jnp.maximum(jnp.sum(dv * dv, axis=1, keepdims=True), EPS)
    term_dir = dv / (W_CONST * d2v1)
    W = Wk + 1.0 / W_CONST
    n_tilde = (term_knn + term_dir) / W
    nrm = jnp.sqrt(jnp.sum(n_tilde * n_tilde, axis=1, keepdims=True))
    nc = n_tilde / (nrm + 1e-8)
    s = jnp.sum(dv * nc, axis=1, keepdims=True)
    xc = x - s * nc
    o_ref[...] = jnp.concatenate([xc, s, nc], axis=1)


def kernel(x, vertices, vertex_normals):
    x = x.astype(jnp.float32)
    vertices = vertices.astype(jnp.float32)
    vertex_normals = vertex_normals.astype(jnp.float32)

    n = x.shape[0]
    v = vertices.shape[0]

    x2 = jnp.sum(x * x, axis=1, keepdims=True)
    xa = jnp.concatenate([x, x2, jnp.zeros((n, 4), jnp.float32)], axis=1)
    v2 = jnp.sum(vertices * vertices, axis=1)
    vt = jnp.concatenate(
        [vertices.T, jnp.zeros((1, v), jnp.float32), v2[None, :], jnp.zeros((3, v), jnp.float32)],
        axis=0,
    )
    R = jnp.concatenate(
        [vertex_normals, jnp.ones((v, 1), jnp.float32), vertices, jnp.zeros((v, 121), jnp.float32)],
        axis=1,
    )

    grid = (n // BN,)
    out = pl.pallas_call(
        _body,
        grid=grid,
        in_specs=[
            pl.BlockSpec((BN, 8), lambda i: (i, 0)),
            pl.BlockSpec((BN, 3), lambda i: (i, 0)),
            pl.BlockSpec((8, N_V), lambda i: (0, 0)),
            pl.BlockSpec((N_V, 128), lambda i: (0, 0)),
        ],
        out_specs=pl.BlockSpec((BN, 7), lambda i: (i, 0)),
        out_shape=jax.ShapeDtypeStruct((n, 7), jnp.float32),
    )(xa, x, vt, R)
    return out


# speedup vs baseline: 8.5142x; 1.5889x over previous
import jax
import jax.numpy as jnp
from jax.experimental import pallas as pl

K = 8
W_CONST = 0.01
EPS = 1e-8
N_V = 16384
BN = 128


def _body(xa_ref, x_ref, vt_ref, Rh_ref, Rl_ref, o_ref):
    xa = xa_ref[...]
    vt = vt_ref[...]
    m = jnp.dot(xa, vt, preferred_element_type=jnp.float32)
    e = vt[4:5, :] + m

    h = N_V // 2
    m0 = jnp.minimum(e[:, :h], e[:, h:])
    s0 = jnp.maximum(e[:, :h], e[:, h:])
    q = h // 2
    m1_ = jnp.minimum(m0[:, :q], m0[:, q:])
    s1 = jnp.minimum(jnp.maximum(m0[:, :q], m0[:, q:]),
                     jnp.minimum(s0[:, :q], s0[:, q:]))
    r = q // 2
    m2 = jnp.minimum(m1_[:, :r], m1_[:, r:])
    s2 = jnp.minimum(jnp.maximum(m1_[:, :r], m1_[:, r:]),
                     jnp.minimum(s1[:, :r], s1[:, r:]))
    cand = jnp.concatenate([m2, s2], axis=1)

    mn = jnp.min(cand, axis=1, keepdims=True)
    m_prev = mn
    for k in range(K - 1):
        cur = jnp.where(cand > m_prev, cand, jnp.inf)
        m_prev = jnp.min(cur, axis=1, keepdims=True)
    t8 = m_prev

    d2 = e + xa[:, 3:4]
    inv = 1.0 / jnp.maximum(d2, EPS)
    w = jnp.where(e <= t8, inv, 0.0)
    onehot = jnp.where(e <= mn, 1.0, 0.0)

    Rh = Rh_ref[...]
    Rl = Rl_ref[...]
    wh = w.astype(jnp.bfloat16)
    wl = (w - wh.astype(jnp.float32)).astype(jnp.bfloat16)
    ohb = onehot.astype(jnp.bfloat16)
    P1 = (jnp.dot(wh, Rh, preferred_element_type=jnp.float32)
          + jnp.dot(wh, Rl, preferred_element_type=jnp.float32)
          + jnp.dot(wl, Rh, preferred_element_type=jnp.float32))
    P2 = (jnp.dot(ohb, Rh, preferred_element_type=jnp.float32)
          + jnp.dot(ohb, Rl, preferred_element_type=jnp.float32))

    term_knn = P1[:, 0:3]
    Wk = P1[:, 3:4]
    cnt = P2[:, 3:4]
    v1 = P2[:, 4:7] / cnt

    x = x_ref[...]
    dv = x - v1
    d2v1 = jnp.maximum(jnp.sum(dv * dv, axis=1, keepdims=True), EPS)
    term_dir = dv / (W_CONST * d2v1)
    W = Wk + 1.0 / W_CONST
    n_tilde = (term_knn + term_dir) / W
    nrm = jnp.sqrt(jnp.sum(n_tilde * n_tilde, axis=1, keepdims=True))
    nc = n_tilde / (nrm + 1e-8)
    s = jnp.sum(dv * nc, axis=1, keepdims=True)
    xc = x - s * nc
    o_ref[...] = jnp.concatenate([xc, s, nc], axis=1)


def kernel(x, vertices, vertex_normals):
    x = x.astype(jnp.float32)
    vertices = vertices.astype(jnp.float32)
    vertex_normals = vertex_normals.astype(jnp.float32)

    n = x.shape[0]
    v = vertices.shape[0]

    x2 = jnp.sum(x * x, axis=1, keepdims=True)
    xa = jnp.concatenate([-2.0 * x, x2, jnp.zeros((n, 4), jnp.float32)], axis=1)
    v2 = jnp.sum(vertices * vertices, axis=1)
    vt = jnp.concatenate(
        [vertices.T, jnp.zeros((1, v), jnp.float32), v2[None, :], jnp.zeros((3, v), jnp.float32)],
        axis=0,
    )
    R = jnp.concatenate(
        [vertex_normals, jnp.ones((v, 1), jnp.float32), vertices, jnp.zeros((v, 121), jnp.float32)],
        axis=1,
    )
    Rh = R.astype(jnp.bfloat16)
    Rl = (R - Rh.astype(jnp.float32)).astype(jnp.bfloat16)

    grid = (n // BN,)
    out = pl.pallas_call(
        _body,
        grid=grid,
        in_specs=[
            pl.BlockSpec((BN, 8), lambda i: (i, 0)),
            pl.BlockSpec((BN, 3), lambda i: (i, 0)),
            pl.BlockSpec((8, N_V), lambda i: (0, 0)),
            pl.BlockSpec((N_V, 128), lambda i: (0, 0)),
            pl.BlockSpec((N_V, 128), lambda i: (0, 0)),
        ],
        out_specs=pl.BlockSpec((BN, 7), lambda i: (i, 0)),
        out_shape=jax.ShapeDtypeStruct((n, 7), jnp.float32),
    )(xa, x, vt, Rh, Rl)
    return out
